# baseline (device time: 12279 ns/iter reference)
import jax
import jax.numpy as jnp
from jax import lax
from jax.experimental import pallas as pl
from jax.experimental.pallas import tpu as pltpu

N_DEV = 16
EPS = 1e-5


def kernel(x, gamma):
    m, n_per = x.shape
    n_global = n_per * N_DEV

    def body(x_hbm, g_ref, out_hbm, xv_ref, outv_ref, partials_ref,
             in_sem, out_sem0, out_sem1, send_sems, recv_sems):
        my_i = lax.axis_index("i")

        in_copy = pltpu.make_async_copy(x_hbm, xv_ref, in_sem)
        in_copy.start()

        barrier_sem = pltpu.get_barrier_semaphore()
        for o in range(1, N_DEV):
            d = (my_i + o) % N_DEV
            pl.semaphore_signal(
                barrier_sem, inc=1,
                device_id=(d,), device_id_type=pl.DeviceIdType.MESH,
            )

        in_copy.wait()

        xv = xv_ref[...]
        ssq = jnp.sum(xv * xv, axis=1)
        partials_ref[my_i, :] = ssq

        pl.semaphore_wait(barrier_sem, N_DEV - 1)

        sends = []
        for o in range(1, N_DEV):
            d = (my_i + o) % N_DEV
            rdma = pltpu.make_async_remote_copy(
                src_ref=partials_ref.at[my_i],
                dst_ref=partials_ref.at[my_i],
                send_sem=send_sems.at[o - 1],
                recv_sem=recv_sems.at[my_i],
                device_id=(d,),
                device_id_type=pl.DeviceIdType.MESH,
            )
            rdma.start()
            sends.append(rdma)

        xg = xv * g_ref[...]

        for o in range(1, N_DEV):
            s = (my_i - o) % N_DEV
            recv = pltpu.make_async_remote_copy(
                src_ref=partials_ref.at[my_i],
                dst_ref=partials_ref.at[s],
                send_sem=send_sems.at[0],
                recv_sem=recv_sems.at[s],
                device_id=(s,),
                device_id_type=pl.DeviceIdType.MESH,
            )
            recv.wait_recv()

        total = jnp.sum(partials_ref[...], axis=0)
        inv = lax.rsqrt(total / n_global + EPS)
        inv_col = inv[:, None]

        h = n_per // 2
        outv_ref[:, :h] = xg[:, :h] * inv_col
        out_copy0 = pltpu.make_async_copy(
            outv_ref.at[:, :h], out_hbm.at[:, :h], out_sem0)
        out_copy0.start()
        outv_ref[:, h:] = xg[:, h:] * inv_col
        out_copy1 = pltpu.make_async_copy(
            outv_ref.at[:, h:], out_hbm.at[:, h:], out_sem1)
        out_copy1.start()

        for rdma in sends:
            rdma.wait_send()
        out_copy0.wait()
        out_copy1.wait()

    return pl.pallas_call(
        body,
        out_shape=jax.ShapeDtypeStruct((m, n_per), x.dtype),
        in_specs=[
            pl.BlockSpec(memory_space=pl.ANY),
            pl.BlockSpec(memory_space=pltpu.VMEM),
        ],
        out_specs=pl.BlockSpec(memory_space=pl.ANY),
        scratch_shapes=[
            pltpu.VMEM((m, n_per), jnp.float32),
            pltpu.VMEM((m, n_per), jnp.float32),
            pltpu.VMEM((N_DEV, m), jnp.float32),
            pltpu.SemaphoreType.DMA,
            pltpu.SemaphoreType.DMA,
            pltpu.SemaphoreType.DMA,
            pltpu.SemaphoreType.DMA((N_DEV - 1,)),
            pltpu.SemaphoreType.DMA((N_DEV,)),
        ],
        compiler_params=pltpu.CompilerParams(collective_id=0),
    )(x, gamma)


# device time: 11777 ns/iter; 1.0426x vs baseline; 1.0426x over previous
import jax
import jax.numpy as jnp
from jax import lax
from jax.experimental import pallas as pl
from jax.experimental.pallas import tpu as pltpu

N_DEV = 16
EPS = 1e-5


def kernel(x, gamma):
    m, n_per = x.shape
    n_global = n_per * N_DEV

    def body(x_ref, g_ref, out_ref, partials_ref, send_sems, recv_sems):
        my_i = lax.axis_index("i")

        barrier_sem = pltpu.get_barrier_semaphore()
        for o in range(1, N_DEV):
            d = (my_i + o) % N_DEV
            pl.semaphore_signal(
                barrier_sem, inc=1,
                device_id=(d,), device_id_type=pl.DeviceIdType.MESH,
            )

        xv = x_ref[...]
        ssq = jnp.sum(xv * xv, axis=1)
        partials_ref[my_i, :] = ssq

        pl.semaphore_wait(barrier_sem, N_DEV - 1)

        sends = []
        for o in range(1, N_DEV):
            d = (my_i + o) % N_DEV
            rdma = pltpu.make_async_remote_copy(
                src_ref=partials_ref.at[my_i],
                dst_ref=partials_ref.at[my_i],
                send_sem=send_sems.at[o - 1],
                recv_sem=recv_sems.at[my_i],
                device_id=(d,),
                device_id_type=pl.DeviceIdType.MESH,
            )
            rdma.start()
            sends.append(rdma)

        xg = xv * g_ref[...]

        for o in range(1, N_DEV):
            s = (my_i - o) % N_DEV
            recv = pltpu.make_async_remote_copy(
                src_ref=partials_ref.at[my_i],
                dst_ref=partials_ref.at[s],
                send_sem=send_sems.at[0],
                recv_sem=recv_sems.at[s],
                device_id=(s,),
                device_id_type=pl.DeviceIdType.MESH,
            )
            recv.wait_recv()

        total = jnp.sum(partials_ref[...], axis=0)
        inv = lax.rsqrt(total / n_global + EPS)
        out_ref[...] = xg * inv[:, None]

        for rdma in sends:
            rdma.wait_send()

    return pl.pallas_call(
        body,
        out_shape=jax.ShapeDtypeStruct((m, n_per), x.dtype),
        in_specs=[
            pl.BlockSpec(memory_space=pltpu.VMEM),
            pl.BlockSpec(memory_space=pltpu.VMEM),
        ],
        out_specs=pl.BlockSpec(memory_space=pltpu.VMEM),
        scratch_shapes=[
            pltpu.VMEM((N_DEV, m), jnp.float32),
            pltpu.SemaphoreType.DMA((N_DEV - 1,)),
            pltpu.SemaphoreType.DMA((N_DEV,)),
        ],
        compiler_params=pltpu.CompilerParams(collective_id=0),
    )(x, gamma.reshape(1, n_per))
